# baseline (speedup 1.0000x reference)
import functools

import jax
import jax.numpy as jnp
from jax import lax
from jax.experimental import pallas as pl
from jax.experimental.pallas import tpu as pltpu
from jax.experimental.pallas import tpu_sc as plsc

_EMBED_DIM = 16
_INPUT_DIM = 64
_HIDDEN = 64
_BATCH = 16384

_NC = 2
_NS = 16
_NW = _NC * _NS
_BPW = _BATCH // _NW
_CHUNK = 128
_NCHUNK = _BPW // _CHUNK


def _sc_gather(table, idx3):
    mesh = plsc.VectorSubcoreMesh(core_axis_name="c", subcore_axis_name="s")

    @functools.partial(
        pl.kernel,
        mesh=mesh,
        compiler_params=pltpu.CompilerParams(use_tc_tiling_on_sc=False),
        out_type=jax.ShapeDtypeStruct((_BATCH, 128), jnp.float32),
        scratch_types=[
            pltpu.VMEM((_NCHUNK, _CHUNK), jnp.int32),
            pltpu.VMEM((_BPW, _EMBED_DIM), jnp.float32),
            pltpu.SemaphoreType.DMA,
        ],
    )
    def gather_kernel(table_hbm, idx_hbm, out_hbm, idx_v, rows_v, sem):
        wid = lax.axis_index("s") * _NC + lax.axis_index("c")
        pltpu.sync_copy(idx_hbm.at[wid], idx_v)
        copies = [
            pltpu.async_copy(
                table_hbm.at[idx_v.at[j]],
                rows_v.at[pl.ds(j * _CHUNK, _CHUNK)],
                sem,
            )
            for j in range(_NCHUNK)
        ]
        for cp in copies:
            cp.wait()
        pltpu.sync_copy(
            rows_v,
            out_hbm.at[pl.ds(wid * _BPW, _BPW), pl.ds(0, _EMBED_DIM)])

    return gather_kernel(table, idx3)


_BR = 2048


def _mlp_body(emb_hbm, feat_ref, w1_ref, b1_ref, w2t_ref, out_ref, emb_vmem,
              sem):
    i = pl.program_id(0)
    cp = pltpu.make_async_copy(emb_hbm.at[pl.ds(i * _BR, _BR), :], emb_vmem,
                               sem)
    cp.start()
    w1e = w1_ref[0:_EMBED_DIM, :]
    w1f = w1_ref[_EMBED_DIM:, :]
    x = jnp.dot(feat_ref[...], w1f, preferred_element_type=jnp.float32)
    cp.wait()
    x = x + jnp.dot(emb_vmem[:, :_EMBED_DIM], w1e,
                    preferred_element_type=jnp.float32)
    h = jnp.maximum(x + b1_ref[...], 0.0)
    o = jnp.sum(h * w2t_ref[...], axis=1, keepdims=True)
    out_ref[...] = jnp.concatenate(
        [o, jnp.zeros((_BR, 127), jnp.float32)], axis=1)


def _tc_mlp(emb128, features, W1, b1r, w2t, interpret=False):
    grid = (_BATCH // _BR,)
    return pl.pallas_call(
        _mlp_body,
        grid=grid,
        in_specs=[
            pl.BlockSpec(memory_space=pl.ANY),
            pl.BlockSpec((_BR, _INPUT_DIM), lambda i: (i, 0)),
            pl.BlockSpec((_EMBED_DIM + _INPUT_DIM, _HIDDEN), lambda i: (0, 0)),
            pl.BlockSpec((1, _HIDDEN), lambda i: (0, 0)),
            pl.BlockSpec((1, _HIDDEN), lambda i: (0, 0)),
        ],
        out_specs=pl.BlockSpec((_BR, 128), lambda i: (i, 0)),
        out_shape=jax.ShapeDtypeStruct((_BATCH, 128), jnp.float32),
        scratch_shapes=[
            pltpu.VMEM((_BR, 128), jnp.float32),
            pltpu.SemaphoreType.DMA,
        ],
        interpret=interpret,
    )(emb128, features, W1, b1r, w2t)


def kernel(pitcher_id, features, table, W1, b1, W2, b2):
    pid = pitcher_id.astype(jnp.int32)
    idx3 = pid.reshape(_NW, _NCHUNK, _CHUNK)
    emb128 = _sc_gather(table, idx3)
    b1r = b1.reshape(1, _HIDDEN)
    w2t = W2.reshape(1, _HIDDEN)
    out128 = _tc_mlp(emb128, features, W1, b1r, w2t)
    return out128[:, :1] + b2
